# baseline (speedup 1.0000x reference)
import functools

import jax
import jax.numpy as jnp
from jax import lax
from jax.experimental import pallas as pl
from jax.experimental.pallas import tpu as pltpu
from jax.experimental.pallas import tpu_sc as plsc

VOCAB = 50257
HIDDEN = 128
SEQ = 2048

_NC, _NS = 2, 16
_NW = _NC * _NS
_B_PER_W = SEQ // _NW

_TV = 1024
_NBUF = 4
_NVB = VOCAB // _TV
_TAIL_BLK = 392


def _embed_gather(tokens, embed_table):
    mesh = plsc.VectorSubcoreMesh(core_axis_name="c", subcore_axis_name="s")

    @functools.partial(
        pl.kernel,
        mesh=mesh,
        out_type=jax.ShapeDtypeStruct((SEQ, HIDDEN), jnp.float32),
        scratch_types=[
            pltpu.VMEM((_B_PER_W,), jnp.int32),
            pltpu.VMEM((_B_PER_W, HIDDEN), jnp.float32),
            pltpu.SemaphoreType.DMA,
        ],
    )
    def gather_kernel(tokens_hbm, table_hbm, out_hbm, idx_v, rows_v, sem):
        wid = lax.axis_index("s") * _NC + lax.axis_index("c")
        base = wid * _B_PER_W
        pltpu.sync_copy(tokens_hbm.at[pl.ds(base, _B_PER_W)], idx_v)
        pltpu.async_copy(table_hbm.at[idx_v], rows_v, sem).wait()
        pltpu.sync_copy(rows_v, out_hbm.at[pl.ds(base, _B_PER_W)])

    return gather_kernel(tokens, embed_table)


def _mlp(x_ref, w1_ref, b1_ref, w2_ref, b2_ref):
    h1 = jnp.maximum(
        jnp.dot(x_ref[...], w1_ref[...],
                preferred_element_type=jnp.float32) + b1_ref[...], 0.0)
    return jnp.maximum(
        jnp.dot(h1, w2_ref[...],
                preferred_element_type=jnp.float32) + b2_ref[...], 0.0)


def _bulk_body(x_ref, w1_ref, b1_ref, w2_ref, b2_ref, wh_ref, bh_ref,
               out_ref, h_ref, bufs_ref, sems):
    i = pl.program_id(0)

    @pl.when(i == 0)
    def _():
        h_ref[...] = _mlp(x_ref, w1_ref, b1_ref, w2_ref, b2_ref)

    slot = lax.rem(i, _NBUF)
    buf = bufs_ref.at[slot]
    sem = sems.at[slot]

    @pl.when(i >= _NBUF)
    def _():
        pltpu.make_async_copy(
            buf, out_ref.at[:, pl.ds((i - _NBUF) * _TV, _TV)], sem).wait()

    buf[...] = jnp.dot(h_ref[...], wh_ref[...],
                       preferred_element_type=jnp.float32) + bh_ref[...]
    pltpu.make_async_copy(
        buf, out_ref.at[:, pl.ds(i * _TV, _TV)], sem).start()

    @pl.when(i == _NVB - 1)
    def _():
        for d in range(_NBUF):
            j = i - (_NBUF - 1) + d
            s = lax.rem(j, _NBUF)
            pltpu.make_async_copy(
                bufs_ref.at[s], out_ref.at[:, pl.ds(j * _TV, _TV)],
                sems.at[s]).wait()


def _tail_body(x_ref, w1_ref, b1_ref, w2_ref, b2_ref, wh_ref, bh_ref,
               alias_ref, out_ref):
    h = _mlp(x_ref, w1_ref, b1_ref, w2_ref, b2_ref)
    out_ref[...] = jnp.dot(h, wh_ref[...],
                           preferred_element_type=jnp.float32) + bh_ref[...]


def kernel(tokens, embed_table, W1, b1, W2, b2, W_head, b_head):
    tokens = tokens.astype(jnp.int32)
    x = _embed_gather(tokens, embed_table)
    b1_2d, b2_2d, bh_2d = b1[None, :], b2[None, :], b_head[None, :]

    bulk = pl.pallas_call(
        _bulk_body,
        grid=(_NVB,),
        in_specs=[
            pl.BlockSpec((SEQ, HIDDEN), lambda i: (0, 0)),
            pl.BlockSpec((HIDDEN, HIDDEN), lambda i: (0, 0)),
            pl.BlockSpec((1, HIDDEN), lambda i: (0, 0)),
            pl.BlockSpec((HIDDEN, HIDDEN), lambda i: (0, 0)),
            pl.BlockSpec((1, HIDDEN), lambda i: (0, 0)),
            pl.BlockSpec((HIDDEN, _TV), lambda i: (0, i)),
            pl.BlockSpec((1, _TV), lambda i: (0, i)),
        ],
        out_specs=pl.BlockSpec(memory_space=pl.ANY),
        out_shape=jax.ShapeDtypeStruct((SEQ, VOCAB), jnp.float32),
        scratch_shapes=[
            pltpu.VMEM((SEQ, HIDDEN), jnp.float32),
            pltpu.VMEM((_NBUF, SEQ, _TV), jnp.float32),
            pltpu.SemaphoreType.DMA((_NBUF,)),
        ],
    )(x, W1, b1_2d, W2, b2_2d, W_head, bh_2d)

    logits = pl.pallas_call(
        _tail_body,
        grid=(1,),
        in_specs=[
            pl.BlockSpec((SEQ, HIDDEN), lambda i: (0, 0)),
            pl.BlockSpec((HIDDEN, HIDDEN), lambda i: (0, 0)),
            pl.BlockSpec((1, HIDDEN), lambda i: (0, 0)),
            pl.BlockSpec((HIDDEN, HIDDEN), lambda i: (0, 0)),
            pl.BlockSpec((1, HIDDEN), lambda i: (0, 0)),
            pl.BlockSpec((HIDDEN, HIDDEN), lambda i: (0, _TAIL_BLK)),
            pl.BlockSpec((1, HIDDEN), lambda i: (0, _TAIL_BLK)),
            pl.BlockSpec(memory_space=pl.ANY),
        ],
        out_specs=pl.BlockSpec((SEQ, HIDDEN), lambda i: (0, _TAIL_BLK)),
        out_shape=jax.ShapeDtypeStruct((SEQ, VOCAB), jnp.float32),
        input_output_aliases={7: 0},
    )(x, W1, b1_2d, W2, b2_2d, W_head, bh_2d, bulk)
    return logits


# speedup vs baseline: 1.0016x; 1.0016x over previous
import functools

import jax
import jax.numpy as jnp
from jax import lax
from jax.experimental import pallas as pl
from jax.experimental.pallas import tpu as pltpu
from jax.experimental.pallas import tpu_sc as plsc

VOCAB = 50257
HIDDEN = 128
SEQ = 2048

_NC, _NS = 2, 16
_NW = _NC * _NS
_B_PER_W = SEQ // _NW

_TV = 1024
_NBUF = 4
_NVB = VOCAB // _TV
_TAIL_BLK = 392


def _embed_gather(tokens, embed_table):
    mesh = plsc.VectorSubcoreMesh(core_axis_name="c", subcore_axis_name="s")

    @functools.partial(
        pl.kernel,
        mesh=mesh,
        out_type=jax.ShapeDtypeStruct((SEQ, HIDDEN), jnp.float32),
        scratch_types=[
            pltpu.VMEM((_B_PER_W,), jnp.int32),
            pltpu.VMEM((_B_PER_W, HIDDEN), jnp.float32),
            pltpu.SemaphoreType.DMA,
        ],
    )
    def gather_kernel(tokens_hbm, table_hbm, out_hbm, idx_v, rows_v, sem):
        wid = lax.axis_index("s") * _NC + lax.axis_index("c")
        base = wid * _B_PER_W
        pltpu.sync_copy(tokens_hbm.at[pl.ds(base, _B_PER_W)], idx_v)
        pltpu.async_copy(table_hbm.at[idx_v], rows_v, sem).wait()
        pltpu.sync_copy(rows_v, out_hbm.at[pl.ds(base, _B_PER_W)])

    return gather_kernel(tokens, embed_table)


def _mlp(x_ref, w1_ref, b1_ref, w2_ref, b2_ref):
    h1 = jnp.maximum(
        jnp.dot(x_ref[...], w1_ref[...],
                preferred_element_type=jnp.float32) + b1_ref[...], 0.0)
    return jnp.maximum(
        jnp.dot(h1, w2_ref[...],
                preferred_element_type=jnp.float32) + b2_ref[...], 0.0)


def _bulk_body(x_ref, w1_ref, b1_ref, w2_ref, b2_ref, wh_ref, bh_ref,
               out_ref, h_ref, bufs_ref, sems):
    i = pl.program_id(0)

    @pl.when(i == 0)
    def _():
        h_ref[...] = _mlp(x_ref, w1_ref, b1_ref, w2_ref, b2_ref)

    slot = lax.rem(i, _NBUF)
    for s in range(_NBUF):
        @pl.when(slot == s)
        def _(s=s):
            buf = bufs_ref.at[s]
            sem = sems.at[s]

            @pl.when(i >= _NBUF)
            def _():
                pltpu.make_async_copy(
                    buf, out_ref.at[:, pl.ds((i - _NBUF) * _TV, _TV)],
                    sem).wait()

            buf[...] = jnp.dot(h_ref[...], wh_ref[...],
                               preferred_element_type=jnp.float32) + bh_ref[...]
            pltpu.make_async_copy(
                buf, out_ref.at[:, pl.ds(i * _TV, _TV)], sem).start()

    @pl.when(i == _NVB - 1)
    def _():
        for d in range(_NBUF):
            j = _NVB - _NBUF + d
            s = j % _NBUF
            pltpu.make_async_copy(
                bufs_ref.at[s], out_ref.at[:, pl.ds(j * _TV, _TV)],
                sems.at[s]).wait()


def _tail_body(x_ref, w1_ref, b1_ref, w2_ref, b2_ref, wh_ref, bh_ref,
               alias_ref, out_ref):
    h = _mlp(x_ref, w1_ref, b1_ref, w2_ref, b2_ref)
    out_ref[...] = jnp.dot(h, wh_ref[...],
                           preferred_element_type=jnp.float32) + bh_ref[...]


def kernel(tokens, embed_table, W1, b1, W2, b2, W_head, b_head):
    tokens = tokens.astype(jnp.int32)
    x = _embed_gather(tokens, embed_table)
    b1_2d, b2_2d, bh_2d = b1[None, :], b2[None, :], b_head[None, :]

    bulk = pl.pallas_call(
        _bulk_body,
        grid=(_NVB,),
        in_specs=[
            pl.BlockSpec((SEQ, HIDDEN), lambda i: (0, 0)),
            pl.BlockSpec((HIDDEN, HIDDEN), lambda i: (0, 0)),
            pl.BlockSpec((1, HIDDEN), lambda i: (0, 0)),
            pl.BlockSpec((HIDDEN, HIDDEN), lambda i: (0, 0)),
            pl.BlockSpec((1, HIDDEN), lambda i: (0, 0)),
            pl.BlockSpec((HIDDEN, _TV), lambda i: (0, i)),
            pl.BlockSpec((1, _TV), lambda i: (0, i)),
        ],
        out_specs=pl.BlockSpec(memory_space=pl.ANY),
        out_shape=jax.ShapeDtypeStruct((SEQ, VOCAB), jnp.float32),
        scratch_shapes=[
            pltpu.VMEM((SEQ, HIDDEN), jnp.float32),
            pltpu.VMEM((_NBUF, SEQ, _TV), jnp.float32),
            pltpu.SemaphoreType.DMA((_NBUF,)),
        ],
    )(x, W1, b1_2d, W2, b2_2d, W_head, bh_2d)

    logits = pl.pallas_call(
        _tail_body,
        grid=(1,),
        in_specs=[
            pl.BlockSpec((SEQ, HIDDEN), lambda i: (0, 0)),
            pl.BlockSpec((HIDDEN, HIDDEN), lambda i: (0, 0)),
            pl.BlockSpec((1, HIDDEN), lambda i: (0, 0)),
            pl.BlockSpec((HIDDEN, HIDDEN), lambda i: (0, 0)),
            pl.BlockSpec((1, HIDDEN), lambda i: (0, 0)),
            pl.BlockSpec((HIDDEN, HIDDEN), lambda i: (0, _TAIL_BLK)),
            pl.BlockSpec((1, HIDDEN), lambda i: (0, _TAIL_BLK)),
            pl.BlockSpec(memory_space=pl.ANY),
        ],
        out_specs=pl.BlockSpec((SEQ, HIDDEN), lambda i: (0, _TAIL_BLK)),
        out_shape=jax.ShapeDtypeStruct((SEQ, VOCAB), jnp.float32),
        input_output_aliases={7: 0},
    )(x, W1, b1_2d, W2, b2_2d, W_head, bh_2d, bulk)
    return logits


# speedup vs baseline: 1.0115x; 1.0099x over previous
import functools

import jax
import jax.numpy as jnp
from jax import lax
from jax.experimental import pallas as pl
from jax.experimental.pallas import tpu as pltpu
from jax.experimental.pallas import tpu_sc as plsc

VOCAB = 50257
HIDDEN = 128
SEQ = 2048

_NC, _NS = 2, 16
_NW = _NC * _NS
_B_PER_W = SEQ // _NW

_TR = 64
_NR = SEQ // _TR


def _embed_gather(tokens, embed_table):
    mesh = plsc.VectorSubcoreMesh(core_axis_name="c", subcore_axis_name="s")

    @functools.partial(
        pl.kernel,
        mesh=mesh,
        out_type=jax.ShapeDtypeStruct((SEQ, HIDDEN), jnp.float32),
        scratch_types=[
            pltpu.VMEM((_B_PER_W,), jnp.int32),
            pltpu.VMEM((_B_PER_W, HIDDEN), jnp.float32),
            pltpu.SemaphoreType.DMA,
        ],
    )
    def gather_kernel(tokens_hbm, table_hbm, out_hbm, idx_v, rows_v, sem):
        wid = lax.axis_index("s") * _NC + lax.axis_index("c")
        base = wid * _B_PER_W
        pltpu.sync_copy(tokens_hbm.at[pl.ds(base, _B_PER_W)], idx_v)
        pltpu.async_copy(table_hbm.at[idx_v], rows_v, sem).wait()
        pltpu.sync_copy(rows_v, out_hbm.at[pl.ds(base, _B_PER_W)])

    return gather_kernel(tokens, embed_table)


def _mlp_head_body(x_ref, w1_ref, b1_ref, w2_ref, b2_ref, wh_ref, bh_ref,
                   out_ref):
    h1 = jnp.maximum(
        jnp.dot(x_ref[...], w1_ref[...],
                preferred_element_type=jnp.float32) + b1_ref[...], 0.0)
    h = jnp.maximum(
        jnp.dot(h1, w2_ref[...],
                preferred_element_type=jnp.float32) + b2_ref[...], 0.0)
    out_ref[...] = jnp.dot(h, wh_ref[...],
                           preferred_element_type=jnp.float32) + bh_ref[...]


def kernel(tokens, embed_table, W1, b1, W2, b2, W_head, b_head):
    tokens = tokens.astype(jnp.int32)
    x = _embed_gather(tokens, embed_table)

    logits = pl.pallas_call(
        _mlp_head_body,
        grid=(_NR,),
        in_specs=[
            pl.BlockSpec((_TR, HIDDEN), lambda i: (i, 0)),
            pl.BlockSpec((HIDDEN, HIDDEN), lambda i: (0, 0)),
            pl.BlockSpec((1, HIDDEN), lambda i: (0, 0)),
            pl.BlockSpec((HIDDEN, HIDDEN), lambda i: (0, 0)),
            pl.BlockSpec((1, HIDDEN), lambda i: (0, 0)),
            pl.BlockSpec((HIDDEN, VOCAB), lambda i: (0, 0)),
            pl.BlockSpec((1, VOCAB), lambda i: (0, 0)),
        ],
        out_specs=pl.BlockSpec((_TR, VOCAB), lambda i: (i, 0)),
        out_shape=jax.ShapeDtypeStruct((SEQ, VOCAB), jnp.float32),
        compiler_params=pltpu.CompilerParams(
            vmem_limit_bytes=63 * 1024 * 1024),
    )(x, W1, b1[None, :], W2, b2[None, :], W_head, b_head[None, :])
    return logits
